# baseline (speedup 1.0000x reference)
import jax
import jax.numpy as jnp
from jax import lax
from jax.experimental import pallas as pl
from jax.experimental.pallas import tpu as pltpu
from jax.experimental.pallas import tpu_sc as plsc

B = 16384
EMB = 64
NCATE = 5
NC = 2
NS = 16
NW = NC * NS
BPW = B // NW
CH = 128
KCH = BPW // CH
RPW = BPW // CH
UCH = 64
UKCH = BPW // UCH


def _din_body(users2, uid2, bid2, cid2, block_hbm, cat_hbm,
              out_u, out_b, out_c,
              uid_v, g_v, bid_v, cid_v, ublk, u_v, b_v, c_v,
              sem_i, sem_u, sem_b, sem_c):
    c = lax.axis_index("c")
    s = lax.axis_index("s")
    w = s * NC + c
    base = w * BPW

    idx_cp = [
        pltpu.async_copy(uid2.at[pl.ds(w * RPW, RPW)], uid_v, sem_i),
        pltpu.async_copy(bid2.at[pl.ds(w * RPW, RPW)], bid_v, sem_i),
    ]
    for j in range(NCATE):
        idx_cp.append(pltpu.async_copy(
            cid2.at[pl.ds(j * (B // CH) + w * RPW, RPW)], cid_v.at[j], sem_i))
    for d in idx_cp:
        d.wait()

    for gg in range(BPW // 16):
        f = 16 * gg
        v = uid_v[f // CH, pl.ds(f % CH, 16)]
        g_v[f // UCH, pl.ds(f % UCH, 16)] = lax.shift_right_logical(v, 1)

    u_cp = []
    for k in range(2):
        u_cp.append(pltpu.async_copy(
            users2.at[g_v.at[k]], ublk.at[k % 2], sem_u))
    b_cp = []
    cat0 = []
    for k in range(KCH):
        rows = pl.ds(k * CH, CH)
        b_cp.append(pltpu.async_copy(
            block_hbm.at[bid_v.at[k]], b_v.at[rows], sem_b))
        cat0.append(pltpu.async_copy(
            cat_hbm.at[cid_v.at[0, k]], c_v.at[rows], sem_c))

    iota = lax.iota(jnp.int32, 16)
    ones = jnp.full((16,), 1, jnp.int32)
    for k in range(UKCH):
        u_cp[k].wait()
        bufsel = jnp.full((16,), k % 2, jnp.int32)
        if k == 0:
            for d in cat0:
                d.wait()
            catj = []
            for j in range(1, NCATE):
                for kk in range(KCH):
                    rows = pl.ds(kk * CH, CH)
                    catj.append(pltpu.async_copy(
                        cat_hbm.at[cid_v.at[j, kk]], c_v.at[rows], sem_c,
                        add=True))
        for m in range(UCH // 16):
            f = k * UCH + 16 * m
            uidvec = uid_v[f // CH, pl.ds(f % CH, 16)]
            parity = lax.rem(uidvec, 2)
            addr0 = parity * EMB
            rows = iota + (16 * m)
            drows = iota + f

            def col_step(col, carry, bufsel=bufsel, rows=rows, drows=drows,
                         addr0=addr0):
                colv = ones * col
                vals = plsc.load_gather(ublk, [bufsel, rows, addr0 + colv])
                plsc.store_scatter(u_v, [drows, colv], vals)
                return carry

            lax.fori_loop(0, EMB, col_step, 0)
        if k + 2 < UKCH:
            u_cp.append(pltpu.async_copy(
                users2.at[g_v.at[k + 2]], ublk.at[k % 2], sem_u))

    pltpu.sync_copy(u_v, out_u.at[pl.ds(base, BPW)])
    for d in b_cp:
        d.wait()
    pltpu.sync_copy(b_v, out_b.at[pl.ds(base, BPW)])
    for d in catj:
        d.wait()
    pltpu.sync_copy(c_v, out_c.at[pl.ds(base, BPW)])


@jax.jit
def _din_sc(users2, uid2, bid2, cid2, block_table, cat_scaled):
    mesh = plsc.VectorSubcoreMesh(core_axis_name="c", subcore_axis_name="s",
                                  num_cores=NC, num_subcores=NS)
    out_t = jax.ShapeDtypeStruct((B, EMB), jnp.float32)
    return pl.kernel(
        _din_body,
        out_type=(out_t, out_t, out_t),
        mesh=mesh,
        compiler_params=pltpu.CompilerParams(use_tc_tiling_on_sc=False,
                                             needs_layout_passes=False),
        scratch_types=[
            pltpu.VMEM((KCH, CH), jnp.int32),
            pltpu.VMEM((UKCH, UCH), jnp.int32),
            pltpu.VMEM((KCH, CH), jnp.int32),
            pltpu.VMEM((NCATE, KCH, CH), jnp.int32),
            pltpu.VMEM((2, UCH, 2 * EMB), jnp.float32),
            pltpu.VMEM((BPW, EMB), jnp.float32),
            pltpu.VMEM((BPW, EMB), jnp.float32),
            pltpu.VMEM((BPW, EMB), jnp.float32),
            pltpu.SemaphoreType.DMA,
            pltpu.SemaphoreType.DMA,
            pltpu.SemaphoreType.DMA,
            pltpu.SemaphoreType.DMA,
        ],
    )(users2, uid2, bid2, cid2, block_table, cat_scaled)


def kernel(user_id, block_id, cate_idx, users_table, block_table,
           category_table):
    one = (user_id[0] * 0 + 1).astype(jnp.float32)
    users2 = users_table.reshape(500000, 2 * EMB) * one
    uid2 = user_id.astype(jnp.int32).reshape(B // CH, CH)
    bid2 = block_id.astype(jnp.int32).reshape(B // CH, CH)
    cid2 = cate_idx.astype(jnp.int32).T.reshape(NCATE * (B // CH), CH)
    cat_scaled = category_table * (1.0 / NCATE)
    u, b, cc = _din_sc(users2, uid2, bid2, cid2, block_table, cat_scaled)
    return jnp.concatenate([u, b, cc], axis=1)


# speedup vs baseline: 1.0905x; 1.0905x over previous
import jax
import jax.numpy as jnp
from jax import lax
from jax.experimental import pallas as pl
from jax.experimental.pallas import tpu as pltpu
from jax.experimental.pallas import tpu_sc as plsc

B = 16384
EMB = 64
NCATE = 5
NC = 2
NS = 16
NW = NC * NS
BPW = B // NW
CH = 128
KCH = BPW // CH
RPW = BPW // CH
UCH = 64
UKCH = BPW // UCH


def _din_body(users2, uid2, bid2, cid2, block_hbm, cat_hbm,
              out_u, out_b, out_c,
              uid_v, g_v, bid_v, cid_v, ublk, b_v, c_v,
              sem_i, sem_u, sem_b, sem_c, sem_o):
    c = lax.axis_index("c")
    s = lax.axis_index("s")
    w = s * NC + c
    base = w * BPW

    idx_cp = [
        pltpu.async_copy(uid2.at[pl.ds(w * RPW, RPW)], uid_v, sem_i),
        pltpu.async_copy(bid2.at[pl.ds(w * RPW, RPW)], bid_v, sem_i),
    ]
    for j in range(NCATE):
        idx_cp.append(pltpu.async_copy(
            cid2.at[pl.ds(j * (B // CH) + w * RPW, RPW)], cid_v.at[j], sem_i))
    for d in idx_cp:
        d.wait()

    for gg in range(BPW // 16):
        f = 16 * gg
        g_v[f // UCH, pl.ds(f % UCH, 16)] = uid_v[f // CH, pl.ds(f % CH, 16)]

    u_cp = []
    for k in range(2):
        u_cp.append(pltpu.async_copy(
            users2.at[g_v.at[k]], ublk.at[k % 2], sem_u))
    b_cp = []
    cat0 = []
    for k in range(KCH):
        rows = pl.ds(k * CH, CH)
        b_cp.append(pltpu.async_copy(
            block_hbm.at[bid_v.at[k]], b_v.at[rows], sem_b))
        cat0.append(pltpu.async_copy(
            cat_hbm.at[cid_v.at[0, k]], c_v.at[rows], sem_c))

    o_cp = []
    for k in range(UKCH):
        u_cp[k].wait()
        if k == 0:
            for d in cat0:
                d.wait()
            catj = []
            for j in range(1, NCATE):
                for kk in range(KCH):
                    rows = pl.ds(kk * CH, CH)
                    catj.append(pltpu.async_copy(
                        cat_hbm.at[cid_v.at[j, kk]], c_v.at[rows], sem_c,
                        add=True))
        o_cp.append(pltpu.async_copy(
            ublk.at[k % 2], out_u.at[pl.ds(base + k * UCH, UCH)], sem_o))
        if k + 2 < UKCH:
            o_cp[k].wait()
            u_cp.append(pltpu.async_copy(
                users2.at[g_v.at[k + 2]], ublk.at[k % 2], sem_u))
    for k in range(UKCH - 2, UKCH):
        o_cp[k].wait()
    for d in b_cp:
        d.wait()
    pltpu.sync_copy(b_v, out_b.at[pl.ds(base, BPW)])
    for d in catj:
        d.wait()
    pltpu.sync_copy(c_v, out_c.at[pl.ds(base, BPW)])


@jax.jit
def _din_sc(users2, uid2, bid2, cid2, block_table, cat_scaled):
    mesh = plsc.VectorSubcoreMesh(core_axis_name="c", subcore_axis_name="s",
                                  num_cores=NC, num_subcores=NS)
    out_t = jax.ShapeDtypeStruct((B, EMB), jnp.float32)
    out_u_t = jax.ShapeDtypeStruct((B, 2 * EMB), jnp.float32)
    return pl.kernel(
        _din_body,
        out_type=(out_u_t, out_t, out_t),
        mesh=mesh,
        compiler_params=pltpu.CompilerParams(use_tc_tiling_on_sc=False,
                                             needs_layout_passes=False),
        scratch_types=[
            pltpu.VMEM((KCH, CH), jnp.int32),
            pltpu.VMEM((UKCH, UCH), jnp.int32),
            pltpu.VMEM((KCH, CH), jnp.int32),
            pltpu.VMEM((NCATE, KCH, CH), jnp.int32),
            pltpu.VMEM((2, UCH, 2 * EMB), jnp.float32),
            pltpu.VMEM((BPW, EMB), jnp.float32),
            pltpu.VMEM((BPW, EMB), jnp.float32),
            pltpu.SemaphoreType.DMA,
            pltpu.SemaphoreType.DMA,
            pltpu.SemaphoreType.DMA,
            pltpu.SemaphoreType.DMA,
            pltpu.SemaphoreType.DMA,
        ],
    )(users2, uid2, bid2, cid2, block_table, cat_scaled)


def kernel(user_id, block_id, cate_idx, users_table, block_table,
           category_table):
    users2 = jnp.pad(users_table, ((0, 0), (0, EMB)))
    uid2 = user_id.astype(jnp.int32).reshape(B // CH, CH)
    bid2 = block_id.astype(jnp.int32).reshape(B // CH, CH)
    cid2 = cate_idx.astype(jnp.int32).T.reshape(NCATE * (B // CH), CH)
    cat_scaled = category_table * (1.0 / NCATE)
    u128, b, cc = _din_sc(users2, uid2, bid2, cid2, block_table, cat_scaled)
    return jnp.concatenate([u128[:, :EMB], b, cc], axis=1)
